# baseline (speedup 1.0000x reference)
import functools

import jax
import jax.numpy as jnp
from jax import lax
from jax.experimental import pallas as pl
from jax.experimental.pallas import tpu as pltpu
from jax.experimental.pallas import tpu_sc as plsc

N = 10000
E = 320000
F = 128
R = 8
SUB = 16

NC = 2
NS = 16
NW = NC * NS
EPW = E // NW
K = 128
EPT = 10240
CH = EPT // K
NSEG = 5
SCH = CH // NSEG
NP = 10240
RPT = NP // NS
ZR = 16
BN = 2000


def _table_body(x_ref, w_ref, out_ref):
    bi = lax.broadcasted_iota(jnp.int32, (F, SUB), 0) // SUB
    cols = []
    for r in range(R):
        w2 = w_ref[r]
        cols.extend(jnp.where(bi == b, w2, 0.0) for b in range(F // SUB))
    bd = jnp.concatenate(cols, axis=1)
    y = jnp.dot(x_ref[...], bd, preferred_element_type=jnp.float32)
    for r in range(R):
        out_ref[r] = y[:, r * F:(r + 1) * F]


def _make_table(x, weight):
    grid = (N // BN,)
    return pl.pallas_call(
        _table_body,
        grid=grid,
        in_specs=[
            pl.BlockSpec((BN, F), lambda n: (n, 0)),
            pl.BlockSpec((R, F, SUB), lambda n: (0, 0, 0)),
        ],
        out_specs=pl.BlockSpec((R, BN, F), lambda n: (0, n, 0)),
        out_shape=jax.ShapeDtypeStruct((R, N, F), jnp.float32),
    )(x, weight.reshape(R, F, SUB))


def _sc_body(table, srcT, etT, dstT, out,
             src_v, idx_v, dst_v, g0, g1, zbuf, acc,
             gs0, gs1, ss0, ss1):
    gbufs = (g0, g1)
    gsems = (gs0, gs1)
    ssems = (ss0, ss1)
    c = lax.axis_index("c")
    s = lax.axis_index("s")
    wid = c * NS + s
    row0 = s * RPT

    zeros16 = jnp.zeros((16,), jnp.float32)

    @pl.loop(0, ZR)
    def _(r):
        for j in range(F // 16):
            zbuf[r, pl.ds(j * 16, 16)] = zeros16

    @pl.loop(0, RPT // ZR)
    def _(i):
        pltpu.async_copy(zbuf, acc.at[pl.ds(row0 + i * ZR, ZR)], gs0)

    @pl.loop(0, RPT // ZR)
    def _(i):
        pltpu.make_async_copy(zbuf, acc.at[pl.ds(row0 + i * ZR, ZR)], gs0).wait()

    plsc.subcore_barrier()

    @pl.loop(0, NSEG)
    def _(seg):
        pltpu.sync_copy(srcT.at[wid, seg], src_v)
        pltpu.sync_copy(etT.at[wid, seg], idx_v)
        pltpu.sync_copy(dstT.at[wid, seg], dst_v)

        @pl.loop(0, SCH)
        def _(g):
            for j in range(K // 16):
                sl = pl.ds(j * 16, 16)
                idx_v[g, sl] = idx_v[g, sl] * N + src_v[g, sl]

        pltpu.async_copy(table.at[idx_v.at[0]], g0, gs0)

        @pl.loop(0, SCH)
        def _(g):
            for b in range(2):
                @pl.when(g % 2 == b)
                def _(b=b):
                    nb = (b + 1) % 2

                    @pl.when(g + 1 < SCH)
                    def _():
                        @pl.when(g >= 1)
                        def _():
                            pltpu.make_async_copy(
                                gbufs[nb], acc.at[dst_v.at[g - 1]],
                                ssems[nb]).wait()
                        pltpu.async_copy(
                            table.at[idx_v.at[g + 1]], gbufs[nb], gsems[nb])

                    pltpu.make_async_copy(
                        table.at[idx_v.at[g]], gbufs[b], gsems[b]).wait()
                    pltpu.async_copy(
                        gbufs[b], acc.at[dst_v.at[g]], ssems[b], add=True)

        pltpu.make_async_copy(
            gbufs[(SCH - 2) % 2], acc.at[dst_v.at[SCH - 2]],
            ssems[(SCH - 2) % 2]).wait()
        pltpu.make_async_copy(
            gbufs[(SCH - 1) % 2], acc.at[dst_v.at[SCH - 1]],
            ssems[(SCH - 1) % 2]).wait()

    plsc.subcore_barrier()

    pltpu.sync_copy(acc.at[pl.ds(row0, RPT)], out.at[c, pl.ds(row0, RPT)])


def _sc_scatter(table, srcT, etT, dstT):
    mesh = plsc.VectorSubcoreMesh(core_axis_name="c", subcore_axis_name="s")
    kern = pl.kernel(
        _sc_body,
        out_type=jax.ShapeDtypeStruct((NC, NP, F), jnp.float32),
        mesh=mesh,
        scratch_types=[
            pltpu.VMEM((SCH, K), jnp.int32),
            pltpu.VMEM((SCH, K), jnp.int32),
            pltpu.VMEM((SCH, K), jnp.int32),
            pltpu.VMEM((K, F), jnp.float32),
            pltpu.VMEM((K, F), jnp.float32),
            pltpu.VMEM((ZR, F), jnp.float32),
            pltpu.VMEM_SHARED((NP, F), jnp.float32),
            pltpu.SemaphoreType.DMA,
            pltpu.SemaphoreType.DMA,
            pltpu.SemaphoreType.DMA,
            pltpu.SemaphoreType.DMA,
        ],
    )
    return kern(table, srcT, etT, dstT)


def _combine_body(p_ref, n_ref, o_ref):
    o_ref[...] = (p_ref[0] + p_ref[1]) * n_ref[...]


def _combine(partial, norm):
    grid = (N // BN,)
    return pl.pallas_call(
        _combine_body,
        grid=grid,
        in_specs=[
            pl.BlockSpec((NC, BN, F), lambda n: (0, n, 0)),
            pl.BlockSpec((BN, 1), lambda n: (n, 0)),
        ],
        out_specs=pl.BlockSpec((BN, F), lambda n: (n, 0)),
        out_shape=jax.ShapeDtypeStruct((N, F), jnp.float32),
    )(partial, norm)


def kernel(x, edge_index, edge_type, norm, weight):
    table = _make_table(x, weight).reshape(R * N, F)
    pad = EPT - EPW
    def _pad_edges(a, fill):
        a2 = jnp.concatenate(
            [a.reshape(NW, EPW),
             jnp.broadcast_to(fill, (NW, pad)).astype(jnp.int32)], axis=1)
        return a2.reshape(NW, NSEG, SCH, K)
    srcT = _pad_edges(edge_index[0], jnp.arange(pad))
    dstT = _pad_edges(edge_index[1], N + jnp.arange(pad) % (NP - N))
    etT = _pad_edges(edge_type, jnp.zeros((pad,), jnp.int32))
    partial = _sc_scatter(table, srcT, etT, dstT)
    return _combine(partial, norm)


# speedup vs baseline: 1.0452x; 1.0452x over previous
import functools

import jax
import jax.numpy as jnp
from jax import lax
from jax.experimental import pallas as pl
from jax.experimental.pallas import tpu as pltpu
from jax.experimental.pallas import tpu_sc as plsc

N = 10000
E = 320000
F = 128
R = 8
SUB = 16

NC = 2
NS = 16
NW = NC * NS
EPW = E // NW
K = 80
CH = EPW // K
NSEG = 5
SCH = CH // NSEG
NP = 10240
RPT = NP // NS
ZR = 16
BN = 2000


def _table_body(x_ref, w_ref, out_ref):
    bi = lax.broadcasted_iota(jnp.int32, (F, SUB), 0) // SUB
    cols = []
    for r in range(R):
        w2 = w_ref[r]
        cols.extend(jnp.where(bi == b, w2, 0.0) for b in range(F // SUB))
    bd = jnp.concatenate(cols, axis=1)
    y = jnp.dot(x_ref[...], bd, preferred_element_type=jnp.float32)
    for r in range(R):
        out_ref[r] = y[:, r * F:(r + 1) * F]


def _make_table(x, weight):
    grid = (N // BN,)
    return pl.pallas_call(
        _table_body,
        grid=grid,
        in_specs=[
            pl.BlockSpec((BN, F), lambda n: (n, 0)),
            pl.BlockSpec((R, F, SUB), lambda n: (0, 0, 0)),
        ],
        out_specs=pl.BlockSpec((R, BN, F), lambda n: (0, n, 0)),
        out_shape=jax.ShapeDtypeStruct((R, N, F), jnp.float32),
    )(x, weight.reshape(R, F, SUB))


def _sc_body(table, srcT, etT, dstT, out,
             src_v, idx_v, dst_v, g0, g1, g2, zbuf, acc,
             gs0, gs1, gs2, ss0, ss1, ss2):
    gbufs = (g0, g1, g2)
    gsems = (gs0, gs1, gs2)
    ssems = (ss0, ss1, ss2)
    KH = K // 2

    def _fire_gather(gi, buf, sem):
        pltpu.async_copy(table.at[idx_v.at[gi, pl.ds(0, KH)]],
                         buf.at[pl.ds(0, KH)], sem)
        pltpu.async_copy(table.at[idx_v.at[gi, pl.ds(KH, KH)]],
                         buf.at[pl.ds(KH, KH)], sem)

    def _wait_gather(gi, buf, sem):
        pltpu.make_async_copy(table.at[idx_v.at[gi]], buf, sem).wait()
    c = lax.axis_index("c")
    s = lax.axis_index("s")
    wid = c * NS + s
    row0 = s * RPT

    zeros16 = jnp.zeros((16,), jnp.float32)

    @pl.loop(0, ZR)
    def _(r):
        for j in range(F // 16):
            zbuf[r, pl.ds(j * 16, 16)] = zeros16

    @pl.loop(0, RPT // ZR)
    def _(i):
        pltpu.async_copy(zbuf, acc.at[pl.ds(row0 + i * ZR, ZR)], gs0)

    @pl.loop(0, RPT // ZR)
    def _(i):
        pltpu.make_async_copy(zbuf, acc.at[pl.ds(row0 + i * ZR, ZR)], gs0).wait()

    plsc.subcore_barrier()

    @pl.loop(0, NSEG)
    def _(seg):
        pltpu.sync_copy(srcT.at[wid, seg], src_v)
        pltpu.sync_copy(etT.at[wid, seg], idx_v)
        pltpu.sync_copy(dstT.at[wid, seg], dst_v)

        @pl.loop(0, SCH)
        def _(g):
            for j in range(K // 16):
                sl = pl.ds(j * 16, 16)
                idx_v[g, sl] = idx_v[g, sl] * N + src_v[g, sl]

        _fire_gather(0, g0, gs0)

        @pl.loop(0, SCH)
        def _(g):
            for b in range(3):
                @pl.when(g % 3 == b)
                def _(b=b):
                    nb = (b + 1) % 3

                    @pl.when(g + 1 < SCH)
                    def _():
                        @pl.when(g >= 2)
                        def _():
                            pltpu.make_async_copy(
                                gbufs[nb], acc.at[dst_v.at[g - 2]],
                                ssems[nb]).wait()
                        _fire_gather(g + 1, gbufs[nb], gsems[nb])

                    _wait_gather(g, gbufs[b], gsems[b])
                    pltpu.async_copy(
                        gbufs[b], acc.at[dst_v.at[g]], ssems[b], add=True)

        pltpu.make_async_copy(
            gbufs[(SCH - 2) % 3], acc.at[dst_v.at[SCH - 2]],
            ssems[(SCH - 2) % 3]).wait()
        pltpu.make_async_copy(
            gbufs[(SCH - 1) % 3], acc.at[dst_v.at[SCH - 1]],
            ssems[(SCH - 1) % 3]).wait()

    plsc.subcore_barrier()

    pltpu.sync_copy(acc.at[pl.ds(row0, RPT)], out.at[c, pl.ds(row0, RPT)])


def _sc_scatter(table, srcT, etT, dstT):
    mesh = plsc.VectorSubcoreMesh(core_axis_name="c", subcore_axis_name="s")
    kern = pl.kernel(
        _sc_body,
        out_type=jax.ShapeDtypeStruct((NC, NP, F), jnp.float32),
        mesh=mesh,
        scratch_types=[
            pltpu.VMEM((SCH, K), jnp.int32),
            pltpu.VMEM((SCH, K), jnp.int32),
            pltpu.VMEM((SCH, K), jnp.int32),
            pltpu.VMEM((K, F), jnp.float32),
            pltpu.VMEM((K, F), jnp.float32),
            pltpu.VMEM((K, F), jnp.float32),
            pltpu.VMEM((ZR, F), jnp.float32),
            pltpu.VMEM_SHARED((NP, F), jnp.float32),
            pltpu.SemaphoreType.DMA,
            pltpu.SemaphoreType.DMA,
            pltpu.SemaphoreType.DMA,
            pltpu.SemaphoreType.DMA,
            pltpu.SemaphoreType.DMA,
            pltpu.SemaphoreType.DMA,
        ],
    )
    return kern(table, srcT, etT, dstT)


def _combine_body(p_ref, n_ref, o_ref):
    o_ref[...] = (p_ref[0] + p_ref[1]) * n_ref[...]


def _combine(partial, norm):
    grid = (N // BN,)
    return pl.pallas_call(
        _combine_body,
        grid=grid,
        in_specs=[
            pl.BlockSpec((NC, BN, F), lambda n: (0, n, 0)),
            pl.BlockSpec((BN, 1), lambda n: (n, 0)),
        ],
        out_specs=pl.BlockSpec((BN, F), lambda n: (n, 0)),
        out_shape=jax.ShapeDtypeStruct((N, F), jnp.float32),
    )(partial, norm)


def kernel(x, edge_index, edge_type, norm, weight):
    table = _make_table(x, weight).reshape(R * N, F)
    srcT = edge_index[0].reshape(NW, NSEG, SCH, K)
    dstT = edge_index[1].reshape(NW, NSEG, SCH, K)
    etT = edge_type.reshape(NW, NSEG, SCH, K)
    partial = _sc_scatter(table, srcT, etT, dstT)
    return _combine(partial, norm)


# speedup vs baseline: 1.0535x; 1.0079x over previous
import functools

import jax
import jax.numpy as jnp
from jax import lax
from jax.experimental import pallas as pl
from jax.experimental.pallas import tpu as pltpu
from jax.experimental.pallas import tpu_sc as plsc

N = 10000
E = 320000
F = 128
R = 8
SUB = 16

NC = 2
NS = 16
NW = NC * NS
EPW = E // NW
K = 80
CH = EPW // K
NSEG = 5
SCH = CH // NSEG
NP = 10240
RPT = NP // NS
ZR = 16
BN = 2000


def _table_body(x_ref, w_ref, out_ref):
    bi = lax.broadcasted_iota(jnp.int32, (F, SUB), 0) // SUB
    cols = []
    for r in range(R):
        w2 = w_ref[r]
        cols.extend(jnp.where(bi == b, w2, 0.0) for b in range(F // SUB))
    bd = jnp.concatenate(cols, axis=1)
    y = jnp.dot(x_ref[...], bd, preferred_element_type=jnp.float32)
    for r in range(R):
        out_ref[r] = y[:, r * F:(r + 1) * F]


def _make_table(x, weight):
    grid = (N // BN,)
    return pl.pallas_call(
        _table_body,
        grid=grid,
        in_specs=[
            pl.BlockSpec((BN, F), lambda n: (n, 0)),
            pl.BlockSpec((R, F, SUB), lambda n: (0, 0, 0)),
        ],
        out_specs=pl.BlockSpec((R, BN, F), lambda n: (0, n, 0)),
        out_shape=jax.ShapeDtypeStruct((R, N, F), jnp.float32),
    )(x, weight.reshape(R, F, SUB))


def _sc_body(table, srcT, etT, dstT, out,
             src_v, idx_v, dst_v, g0, g1, g2, zbuf, acc,
             gs0, gs1, gs2, ss0, ss1, ss2):
    gbufs = (g0, g1, g2)
    gsems = (gs0, gs1, gs2)
    ssems = (ss0, ss1, ss2)
    def _fire_gather(gi, b):
        pltpu.async_copy(table.at[idx_v.at[gi]], gbufs[b], gsems[b])

    def _wait_gather(gi, b):
        pltpu.make_async_copy(table.at[idx_v.at[gi]], gbufs[b], gsems[b]).wait()

    def _fire_scatter(gi, b):
        pltpu.async_copy(gbufs[b], acc.at[dst_v.at[gi]], ssems[b], add=True)

    def _drain_scatter(gi, b):
        pltpu.make_async_copy(gbufs[b], acc.at[dst_v.at[gi]], ssems[b]).wait()
    c = lax.axis_index("c")
    s = lax.axis_index("s")
    wid = c * NS + s
    row0 = s * RPT

    zeros16 = jnp.zeros((16,), jnp.float32)

    @pl.loop(0, ZR)
    def _(r):
        for j in range(F // 16):
            zbuf[r, pl.ds(j * 16, 16)] = zeros16

    @pl.loop(0, RPT // ZR)
    def _(i):
        pltpu.async_copy(zbuf, acc.at[pl.ds(row0 + i * ZR, ZR)], gs0)

    @pl.loop(0, RPT // ZR)
    def _(i):
        pltpu.make_async_copy(zbuf, acc.at[pl.ds(row0 + i * ZR, ZR)], gs0).wait()

    plsc.subcore_barrier()

    @pl.loop(0, NSEG)
    def _(seg):
        pltpu.sync_copy(srcT.at[wid, seg], src_v)
        pltpu.sync_copy(etT.at[wid, seg], idx_v)
        pltpu.sync_copy(dstT.at[wid, seg], dst_v)

        @pl.loop(0, SCH)
        def _(g):
            for j in range(K // 16):
                sl = pl.ds(j * 16, 16)
                idx_v[g, sl] = idx_v[g, sl] * N + src_v[g, sl]

        _fire_gather(0, 0)
        _fire_gather(1, 1)
        _wait_gather(0, 0)
        _fire_scatter(0, 0)
        _fire_gather(2, 2)
        _wait_gather(1, 1)
        _fire_scatter(1, 1)
        _drain_scatter(0, 0)
        _fire_gather(3, 0)
        _wait_gather(2, 2)
        _fire_scatter(2, 2)

        @pl.loop(3, SCH - 1, step=3)
        def _(g):
            for u in range(3):
                c = g + u
                b = u
                nb = (u + 1) % 3
                _drain_scatter(c - 2, nb)
                _fire_gather(c + 1, nb)
                _wait_gather(c, b)
                _fire_scatter(c, b)

        _drain_scatter(SCH - 3, (SCH - 3) % 3)
        _wait_gather(SCH - 1, (SCH - 1) % 3)
        _fire_scatter(SCH - 1, (SCH - 1) % 3)
        _drain_scatter(SCH - 2, (SCH - 2) % 3)
        _drain_scatter(SCH - 1, (SCH - 1) % 3)

    plsc.subcore_barrier()

    pltpu.sync_copy(acc.at[pl.ds(row0, RPT)], out.at[c, pl.ds(row0, RPT)])


def _sc_scatter(table, srcT, etT, dstT):
    mesh = plsc.VectorSubcoreMesh(core_axis_name="c", subcore_axis_name="s")
    kern = pl.kernel(
        _sc_body,
        out_type=jax.ShapeDtypeStruct((NC, NP, F), jnp.float32),
        mesh=mesh,
        scratch_types=[
            pltpu.VMEM((SCH, K), jnp.int32),
            pltpu.VMEM((SCH, K), jnp.int32),
            pltpu.VMEM((SCH, K), jnp.int32),
            pltpu.VMEM((K, F), jnp.float32),
            pltpu.VMEM((K, F), jnp.float32),
            pltpu.VMEM((K, F), jnp.float32),
            pltpu.VMEM((ZR, F), jnp.float32),
            pltpu.VMEM_SHARED((NP, F), jnp.float32),
            pltpu.SemaphoreType.DMA,
            pltpu.SemaphoreType.DMA,
            pltpu.SemaphoreType.DMA,
            pltpu.SemaphoreType.DMA,
            pltpu.SemaphoreType.DMA,
            pltpu.SemaphoreType.DMA,
        ],
    )
    return kern(table, srcT, etT, dstT)


def _combine_body(p_ref, n_ref, o_ref):
    o_ref[...] = (p_ref[0] + p_ref[1]) * n_ref[...]


def _combine(partial, norm):
    grid = (N // BN,)
    return pl.pallas_call(
        _combine_body,
        grid=grid,
        in_specs=[
            pl.BlockSpec((NC, BN, F), lambda n: (0, n, 0)),
            pl.BlockSpec((BN, 1), lambda n: (n, 0)),
        ],
        out_specs=pl.BlockSpec((BN, F), lambda n: (n, 0)),
        out_shape=jax.ShapeDtypeStruct((N, F), jnp.float32),
    )(partial, norm)


def kernel(x, edge_index, edge_type, norm, weight):
    table = _make_table(x, weight).reshape(R * N, F)
    srcT = edge_index[0].reshape(NW, NSEG, SCH, K)
    dstT = edge_index[1].reshape(NW, NSEG, SCH, K)
    etT = edge_type.reshape(NW, NSEG, SCH, K)
    partial = _sc_scatter(table, srcT, etT, dstT)
    return _combine(partial, norm)
